# baseline (speedup 1.0000x reference)
import functools

import jax
import jax.numpy as jnp
from jax import lax
from jax.experimental import pallas as pl
from jax.experimental.pallas import tpu as pltpu
from jax.experimental.pallas import tpu_sc as plsc

N_NODES = 10000
D_FEAT = 128
N_EDGES = 320000
BATCH = 4096

QUARTER = 2544
N_LOC = 2560
TRASH = 2552
ROWS_PT = N_LOC // 16
CHUNK = 128
E_PAD = 327680
CH_PT = E_PAD // 16 // CHUNK
E_PT = E_PAD // 16
SRC_BITS = 14


def _segment_mean_sc(xfeat, edges3, zstripe):
    mesh = plsc.VectorSubcoreMesh(core_axis_name="c", subcore_axis_name="s")

    @functools.partial(
        pl.kernel,
        mesh=mesh,
        compiler_params=pltpu.CompilerParams(needs_layout_passes=False),
        out_type=jax.ShapeDtypeStruct((2, 2, N_LOC, D_FEAT), jnp.float32),
        scratch_types=[
            pltpu.VMEM((CH_PT, CHUNK), jnp.int32),
            pltpu.VMEM((E_PT + 2 * CHUNK,), jnp.int32),
            pltpu.VMEM((CHUNK,), jnp.int32),
            pltpu.VMEM((CHUNK,), jnp.int32),
            pltpu.VMEM((CHUNK,), jnp.int32),
            pltpu.VMEM((CHUNK,), jnp.int32),
            pltpu.VMEM((CHUNK, D_FEAT), jnp.float32),
            pltpu.VMEM((CHUNK, D_FEAT), jnp.float32),
            pltpu.VMEM((N_LOC,), jnp.float32),
            pltpu.VMEM((ROWS_PT,), jnp.float32),
            pltpu.VMEM((ROWS_PT, D_FEAT), jnp.float32),
            pltpu.VMEM_SHARED((N_LOC, D_FEAT), jnp.float32),
            pltpu.VMEM_SHARED((16 * N_LOC,), jnp.float32),
            pltpu.SemaphoreType.DMA,
            pltpu.SemaphoreType.DMA,
            pltpu.SemaphoreType.DMA,
            pltpu.SemaphoreType.DMA,
        ],
    )
    def k(x_h, edge_h, z_h, out_h,
          edge_v, pack_v, sloc_a, dloc_a, sloc_b, dloc_b,
          rows_a, rows_b, hist_v, tbuf_v, sbuf_v,
          accum, stag, sem_a, sem_b, sem_c, sem_d):
        c = lax.axis_index("c")
        s = lax.axis_index("s")
        zero16 = jnp.zeros((16,), jnp.float32)
        one16 = jnp.full((16,), 1.0, jnp.float32)

        pltpu.sync_copy(edge_h.at[s], edge_v)

        for p in range(2):
            base = (c * 2 + p) * QUARTER

            def zh(i, carry):
                hist_v[pl.ds(i * 16, 16)] = zero16
                return carry

            lax.fori_loop(0, N_LOC // 16, zh, 0)
            pltpu.sync_copy(z_h, accum.at[pl.ds(s * ROWS_PT, ROWS_PT)])
            plsc.subcore_barrier()

            def scan(i, off):
                for kk in range(CHUNK // 16):
                    sl = pl.ds(kk * 16, 16)
                    e16 = edge_v[i, sl]
                    d16 = lax.shift_right_logical(e16, SRC_BITS) - base
                    ok = (d16 >= 0) & (d16 < QUARTER)
                    dl = jnp.where(ok, d16, TRASH)
                    pk = (e16 & ((1 << SRC_BITS) - 1)) | (dl << SRC_BITS)
                    plsc.store_compressed(pack_v.at[pl.ds(off, 16)], pk,
                                          mask=ok)
                    plsc.addupdate_scatter(hist_v, [dl], one16, mask=ok)
                    cnt = plsc.all_reduce_population_count(ok)
                    off = off + cnt[0]
                return off

            off = lax.fori_loop(0, CH_PT, scan, jnp.int32(0))
            padv = jnp.full((16,), TRASH << SRC_BITS, jnp.int32)
            for kk in range(2 * CHUNK // 16):
                pack_v[pl.ds(off + kk * 16, 16)] = padv
            ncraw = (off + CHUNK - 1) // CHUNK
            nc2 = ((ncraw + 1) // 2) * 2

            def unpack(i, sloc, dloc):
                for kk in range(CHUNK // 16):
                    sl = pl.ds(kk * 16, 16)
                    pk = pack_v[pl.ds(i * CHUNK + kk * 16, 16)]
                    sloc[sl] = pk & ((1 << SRC_BITS) - 1)
                    dloc[sl] = lax.shift_right_logical(pk, SRC_BITS)

            def body(g, carry):
                unpack(2 * g, sloc_a, dloc_a)
                unpack(2 * g + 1, sloc_b, dloc_b)
                ha = pltpu.async_copy(x_h.at[sloc_a], rows_a, sem_a)
                hb = pltpu.async_copy(x_h.at[sloc_b], rows_b, sem_b)
                ha.wait()
                pltpu.sync_copy(rows_a, accum.at[dloc_a], add=True)
                hb.wait()
                pltpu.sync_copy(rows_b, accum.at[dloc_b], add=True)
                return carry

            lax.fori_loop(0, nc2 // 2, body, 0)

            pltpu.sync_copy(hist_v, stag.at[pl.ds(s * N_LOC, N_LOC)])
            plsc.subcore_barrier()

            def zt(i, carry):
                tbuf_v[pl.ds(i * 16, 16)] = zero16
                return carry

            lax.fori_loop(0, ROWS_PT // 16, zt, 0)
            for t in range(16):
                pltpu.sync_copy(
                    stag.at[pl.ds(t * N_LOC + s * ROWS_PT, ROWS_PT)],
                    hist_v.at[pl.ds(0, ROWS_PT)])

                def acc(i, carry):
                    tbuf_v[pl.ds(i * 16, 16)] = (tbuf_v[pl.ds(i * 16, 16)] +
                                                 hist_v[pl.ds(i * 16, 16)])
                    return carry

                lax.fori_loop(0, ROWS_PT // 16, acc, 0)

            def inv(i, carry):
                d = tbuf_v[pl.ds(i * 16, 16)]
                tbuf_v[pl.ds(i * 16, 16)] = 1.0 / jnp.maximum(d, 1.0)
                return carry

            lax.fori_loop(0, ROWS_PT // 16, inv, 0)

            pltpu.sync_copy(accum.at[pl.ds(s * ROWS_PT, ROWS_PT)], sbuf_v)

            def scale(rg, carry):
                iv16 = tbuf_v[pl.ds(rg * 16, 16)]
                for j in range(16):
                    r = rg * 16 + j
                    iv = iv16[j]
                    for kk in range(D_FEAT // 16):
                        sl = pl.ds(kk * 16, 16)
                        sbuf_v[r, sl] = sbuf_v[r, sl] * iv
                return carry

            lax.fori_loop(0, ROWS_PT // 16, scale, 0)
            pltpu.sync_copy(sbuf_v,
                            out_h.at[c, p, pl.ds(s * ROWS_PT, ROWS_PT)])
            plsc.subcore_barrier()

    return k(xfeat, edges3, zstripe)


def _encode_body(nm_ref, x_ref, w_ref, o_ref):
    wm = w_ref[...]
    w1 = wm[:, :D_FEAT]
    w2 = wm[:, D_FEAT:]
    h = lax.dot_general(x_ref[...], w1, (((1,), (1,)), ((), ())),
                        preferred_element_type=jnp.float32)
    h = h + lax.dot_general(nm_ref[...], w2, (((1,), (1,)), ((), ())),
                            preferred_element_type=jnp.float32)
    o_ref[...] = jnp.maximum(h, 0.0)


def _encode_tc(nm, x, W):
    br = 400
    grid = N_NODES // br
    return pl.pallas_call(
        _encode_body,
        grid=(grid,),
        in_specs=[
            pl.BlockSpec((br, D_FEAT), lambda i: (i, 0)),
            pl.BlockSpec((br, D_FEAT), lambda i: (i, 0)),
            pl.BlockSpec((D_FEAT, 2 * D_FEAT), lambda i: (0, 0)),
        ],
        out_specs=pl.BlockSpec((br, D_FEAT), lambda i: (i, 0)),
        out_shape=jax.ShapeDtypeStruct((N_NODES, D_FEAT), jnp.float32),
    )(nm, x, W)


def _gather_sc(ht, nuv3):
    mesh = plsc.VectorSubcoreMesh(core_axis_name="c", subcore_axis_name="s")
    rows_per_w = 2 * BATCH // 32

    @functools.partial(
        pl.kernel,
        mesh=mesh,
        out_type=jax.ShapeDtypeStruct((2 * BATCH, D_FEAT), jnp.float32),
        scratch_types=[
            pltpu.VMEM((rows_per_w // CHUNK, CHUNK), jnp.int32),
            pltpu.VMEM((rows_per_w, D_FEAT), jnp.float32),
            pltpu.SemaphoreType.DMA,
        ],
    )
    def k(ht_h, idx_h, out_h, idx_v, rows_v, sem):
        c = lax.axis_index("c")
        s = lax.axis_index("s")
        w = c * 16 + s
        pltpu.sync_copy(idx_h.at[w], idx_v)
        for j in range(rows_per_w // CHUNK):
            pltpu.async_copy(ht_h.at[idx_v.at[j]],
                             rows_v.at[pl.ds(j * CHUNK, CHUNK)], sem).wait()
        pltpu.sync_copy(rows_v, out_h.at[pl.ds(w * rows_per_w, rows_per_w)])

    return k(ht, nuv3)


def _cosine_body(u_ref, v_ref, o_ref):
    u = u_ref[...]
    v = v_ref[...]
    num = jnp.sum(u * v, axis=1, keepdims=True)
    uu = jnp.sum(u * u, axis=1, keepdims=True)
    vv = jnp.sum(v * v, axis=1, keepdims=True)
    o_ref[...] = num / jnp.maximum(jnp.sqrt(uu) * jnp.sqrt(vv), 1e-8)


def _cosine_tc(eu, ev):
    br = 512
    grid = BATCH // br
    return pl.pallas_call(
        _cosine_body,
        grid=(grid,),
        in_specs=[
            pl.BlockSpec((br, D_FEAT), lambda i: (i, 0)),
            pl.BlockSpec((br, D_FEAT), lambda i: (i, 0)),
        ],
        out_specs=pl.BlockSpec((br, 1), lambda i: (i, 0)),
        out_shape=jax.ShapeDtypeStruct((BATCH, 1), jnp.float32),
    )(eu, ev)


def kernel(x, edge_index, node_u, node_v, W):
    src = edge_index[0]
    dst = edge_index[1]
    pad = E_PAD - N_EDGES
    src_p = jnp.concatenate([src, jnp.zeros((pad,), jnp.int32)])
    dst_p = jnp.concatenate([dst, jnp.full((pad,), (1 << 17) - 1, jnp.int32)])
    edges3 = (src_p | (dst_p << SRC_BITS)).reshape(16, CH_PT, CHUNK)
    zstripe = jnp.zeros((ROWS_PT, D_FEAT), jnp.float32)

    nm_loc = _segment_mean_sc(x, edges3, zstripe)
    nm = jnp.concatenate([nm_loc[0, 0, :QUARTER], nm_loc[0, 1, :QUARTER],
                          nm_loc[1, 0, :QUARTER],
                          nm_loc[1, 1, :N_NODES - 3 * QUARTER]])
    assert 3 * QUARTER < N_NODES <= 4 * QUARTER
    ht = _encode_tc(nm, x, W)

    nuv3 = jnp.concatenate([node_u, node_v]).reshape(32, -1, CHUNK)
    euv = _gather_sc(ht, nuv3)
    cos2 = _cosine_tc(euv[:BATCH], euv[BATCH:])
    return cos2[:, 0]


# speedup vs baseline: 1.3477x; 1.3477x over previous
import functools

import jax
import jax.numpy as jnp
from jax import lax
from jax.experimental import pallas as pl
from jax.experimental.pallas import tpu as pltpu
from jax.experimental.pallas import tpu_sc as plsc

N_NODES = 10000
D_FEAT = 128
N_EDGES = 320000
BATCH = 4096

QUARTER = 2544
N_LOC = 2560
TRASH = 2552
ROWS_PT = N_LOC // 16
CHUNK = 128
E_PAD = 327680
CH_PT = E_PAD // 16 // CHUNK
E_PT = E_PAD // 16
SRC_BITS = 14


def _segment_mean_sc(xfeat, edges3, zstripe):
    mesh = plsc.VectorSubcoreMesh(core_axis_name="c", subcore_axis_name="s")

    @functools.partial(
        pl.kernel,
        mesh=mesh,
        compiler_params=pltpu.CompilerParams(needs_layout_passes=False),
        out_type=jax.ShapeDtypeStruct((2, 2, N_LOC, D_FEAT), jnp.float32),
        scratch_types=[
            pltpu.VMEM((CH_PT, CHUNK), jnp.int32),
            pltpu.VMEM((E_PT + 2 * CHUNK,), jnp.int32),
            pltpu.VMEM((CHUNK,), jnp.int32),
            pltpu.VMEM((CHUNK,), jnp.int32),
            pltpu.VMEM((CHUNK,), jnp.int32),
            pltpu.VMEM((CHUNK,), jnp.int32),
            pltpu.VMEM((CHUNK, D_FEAT), jnp.float32),
            pltpu.VMEM((CHUNK, D_FEAT), jnp.float32),
            pltpu.VMEM((N_LOC,), jnp.float32),
            pltpu.VMEM((ROWS_PT,), jnp.float32),
            pltpu.VMEM((ROWS_PT, D_FEAT), jnp.float32),
            pltpu.VMEM_SHARED((N_LOC, D_FEAT), jnp.float32),
            pltpu.VMEM_SHARED((16 * N_LOC,), jnp.float32),
            pltpu.SemaphoreType.DMA,
            pltpu.SemaphoreType.DMA,
            pltpu.SemaphoreType.DMA,
            pltpu.SemaphoreType.DMA,
        ],
    )
    def k(x_h, edge_h, z_h, out_h,
          edge_v, pack_v, sloc_a, dloc_a, sloc_b, dloc_b,
          rows_a, rows_b, hist_v, tbuf_v, sbuf_v,
          accum, stag, sem_a, sem_b, sem_c, sem_d):
        c = lax.axis_index("c")
        s = lax.axis_index("s")
        zero16 = jnp.zeros((16,), jnp.float32)
        one16 = jnp.full((16,), 1.0, jnp.float32)

        pltpu.sync_copy(edge_h.at[s], edge_v)

        for p in range(2):
            base = (c * 2 + p) * QUARTER

            def zh(i, carry):
                hist_v[pl.ds(i * 16, 16)] = zero16
                return carry

            lax.fori_loop(0, N_LOC // 16, zh, 0)
            pltpu.sync_copy(z_h, accum.at[pl.ds(s * ROWS_PT, ROWS_PT)])
            plsc.subcore_barrier()

            def scan(i, off):
                for kk in range(CHUNK // 16):
                    sl = pl.ds(kk * 16, 16)
                    e16 = edge_v[i, sl]
                    d16 = lax.shift_right_logical(e16, SRC_BITS) - base
                    ok = (d16 >= 0) & (d16 < QUARTER)
                    dl = jnp.where(ok, d16, TRASH)
                    pk = (e16 & ((1 << SRC_BITS) - 1)) | (dl << SRC_BITS)
                    plsc.store_compressed(pack_v.at[pl.ds(off, 16)], pk,
                                          mask=ok)
                    plsc.addupdate_scatter(hist_v, [dl], one16, mask=ok)
                    cnt = plsc.all_reduce_population_count(ok)
                    off = off + cnt[0]
                return off

            off = lax.fori_loop(0, CH_PT, scan, jnp.int32(0))
            padv = jnp.full((16,), TRASH << SRC_BITS, jnp.int32)
            for kk in range(2 * CHUNK // 16):
                pack_v[pl.ds(off + kk * 16, 16)] = padv
            nchunks = (off + CHUNK - 1) // CHUNK

            def unpack(i, sloc, dloc):
                for kk in range(CHUNK // 16):
                    sl = pl.ds(kk * 16, 16)
                    pk = pack_v[pl.ds(i * CHUNK + kk * 16, 16)]
                    sloc[sl] = pk & ((1 << SRC_BITS) - 1)
                    dloc[sl] = lax.shift_right_logical(pk, SRC_BITS)

            def body(i, carry):
                unpack(i, sloc_a, dloc_a)
                pltpu.async_copy(x_h.at[sloc_a], rows_a, sem_a).wait()
                pltpu.sync_copy(rows_a, accum.at[dloc_a], add=True)
                return carry

            lax.fori_loop(0, nchunks, body, 0)

            pltpu.sync_copy(hist_v, stag.at[pl.ds(s * N_LOC, N_LOC)])
            plsc.subcore_barrier()

            def zt(i, carry):
                tbuf_v[pl.ds(i * 16, 16)] = zero16
                return carry

            lax.fori_loop(0, ROWS_PT // 16, zt, 0)
            for t in range(16):
                pltpu.sync_copy(
                    stag.at[pl.ds(t * N_LOC + s * ROWS_PT, ROWS_PT)],
                    hist_v.at[pl.ds(0, ROWS_PT)])

                def acc(i, carry):
                    tbuf_v[pl.ds(i * 16, 16)] = (tbuf_v[pl.ds(i * 16, 16)] +
                                                 hist_v[pl.ds(i * 16, 16)])
                    return carry

                lax.fori_loop(0, ROWS_PT // 16, acc, 0)

            def inv(i, carry):
                d = tbuf_v[pl.ds(i * 16, 16)]
                tbuf_v[pl.ds(i * 16, 16)] = 1.0 / jnp.maximum(d, 1.0)
                return carry

            lax.fori_loop(0, ROWS_PT // 16, inv, 0)

            pltpu.sync_copy(accum.at[pl.ds(s * ROWS_PT, ROWS_PT)], sbuf_v)

            def scale(rg, carry):
                iv16 = tbuf_v[pl.ds(rg * 16, 16)]
                for j in range(16):
                    r = rg * 16 + j
                    iv = iv16[j]
                    for kk in range(D_FEAT // 16):
                        sl = pl.ds(kk * 16, 16)
                        sbuf_v[r, sl] = sbuf_v[r, sl] * iv
                return carry

            lax.fori_loop(0, ROWS_PT // 16, scale, 0)
            pltpu.sync_copy(sbuf_v,
                            out_h.at[c, p, pl.ds(s * ROWS_PT, ROWS_PT)])
            plsc.subcore_barrier()

    return k(xfeat, edges3, zstripe)


def _encode_body(nm_ref, x_ref, w_ref, o_ref):
    wm = w_ref[...]
    w1 = wm[:, :D_FEAT]
    w2 = wm[:, D_FEAT:]
    h = lax.dot_general(x_ref[...], w1, (((1,), (1,)), ((), ())),
                        preferred_element_type=jnp.float32)
    h = h + lax.dot_general(nm_ref[...], w2, (((1,), (1,)), ((), ())),
                            preferred_element_type=jnp.float32)
    o_ref[...] = jnp.maximum(h, 0.0)


def _encode_tc(nm, x, W):
    br = 400
    grid = N_NODES // br
    return pl.pallas_call(
        _encode_body,
        grid=(grid,),
        in_specs=[
            pl.BlockSpec((br, D_FEAT), lambda i: (i, 0)),
            pl.BlockSpec((br, D_FEAT), lambda i: (i, 0)),
            pl.BlockSpec((D_FEAT, 2 * D_FEAT), lambda i: (0, 0)),
        ],
        out_specs=pl.BlockSpec((br, D_FEAT), lambda i: (i, 0)),
        out_shape=jax.ShapeDtypeStruct((N_NODES, D_FEAT), jnp.float32),
    )(nm, x, W)


def _gather_sc(ht, nuv3):
    mesh = plsc.VectorSubcoreMesh(core_axis_name="c", subcore_axis_name="s")
    rows_per_w = 2 * BATCH // 32

    @functools.partial(
        pl.kernel,
        mesh=mesh,
        out_type=jax.ShapeDtypeStruct((2 * BATCH, D_FEAT), jnp.float32),
        scratch_types=[
            pltpu.VMEM((rows_per_w // CHUNK, CHUNK), jnp.int32),
            pltpu.VMEM((rows_per_w, D_FEAT), jnp.float32),
            pltpu.SemaphoreType.DMA,
        ],
    )
    def k(ht_h, idx_h, out_h, idx_v, rows_v, sem):
        c = lax.axis_index("c")
        s = lax.axis_index("s")
        w = c * 16 + s
        pltpu.sync_copy(idx_h.at[w], idx_v)
        for j in range(rows_per_w // CHUNK):
            pltpu.async_copy(ht_h.at[idx_v.at[j]],
                             rows_v.at[pl.ds(j * CHUNK, CHUNK)], sem).wait()
        pltpu.sync_copy(rows_v, out_h.at[pl.ds(w * rows_per_w, rows_per_w)])

    return k(ht, nuv3)


def _cosine_body(u_ref, v_ref, o_ref):
    u = u_ref[...]
    v = v_ref[...]
    num = jnp.sum(u * v, axis=1, keepdims=True)
    uu = jnp.sum(u * u, axis=1, keepdims=True)
    vv = jnp.sum(v * v, axis=1, keepdims=True)
    o_ref[...] = num / jnp.maximum(jnp.sqrt(uu) * jnp.sqrt(vv), 1e-8)


def _cosine_tc(eu, ev):
    br = 512
    grid = BATCH // br
    return pl.pallas_call(
        _cosine_body,
        grid=(grid,),
        in_specs=[
            pl.BlockSpec((br, D_FEAT), lambda i: (i, 0)),
            pl.BlockSpec((br, D_FEAT), lambda i: (i, 0)),
        ],
        out_specs=pl.BlockSpec((br, 1), lambda i: (i, 0)),
        out_shape=jax.ShapeDtypeStruct((BATCH, 1), jnp.float32),
    )(eu, ev)


def kernel(x, edge_index, node_u, node_v, W):
    src = edge_index[0]
    dst = edge_index[1]
    pad = E_PAD - N_EDGES
    src_p = jnp.concatenate([src, jnp.zeros((pad,), jnp.int32)])
    dst_p = jnp.concatenate([dst, jnp.full((pad,), (1 << 17) - 1, jnp.int32)])
    edges3 = (src_p | (dst_p << SRC_BITS)).reshape(16, CH_PT, CHUNK)
    zstripe = jnp.zeros((ROWS_PT, D_FEAT), jnp.float32)

    nm_loc = _segment_mean_sc(x, edges3, zstripe)
    nm = jnp.concatenate([nm_loc[0, 0, :QUARTER], nm_loc[0, 1, :QUARTER],
                          nm_loc[1, 0, :QUARTER],
                          nm_loc[1, 1, :N_NODES - 3 * QUARTER]])
    assert 3 * QUARTER < N_NODES <= 4 * QUARTER
    ht = _encode_tc(nm, x, W)

    nuv3 = jnp.concatenate([node_u, node_v]).reshape(32, -1, CHUNK)
    euv = _gather_sc(ht, nuv3)
    cos2 = _cosine_tc(euv[:BATCH], euv[BATCH:])
    return cos2[:, 0]


# speedup vs baseline: 1.3483x; 1.0004x over previous
import functools

import jax
import jax.numpy as jnp
from jax import lax
from jax.experimental import pallas as pl
from jax.experimental.pallas import tpu as pltpu
from jax.experimental.pallas import tpu_sc as plsc

N_NODES = 10000
D_FEAT = 128
N_EDGES = 320000
BATCH = 4096

QUARTER = 2544
N_LOC = 2560
TRASH = 2552
ROWS_PT = N_LOC // 16
CHUNK = 128
E_PAD = 327680
CH_PT = E_PAD // 16 // CHUNK
E_PT = E_PAD // 16
SRC_BITS = 14


def _segment_mean_sc(xfeat, edges3, zstripe):
    mesh = plsc.VectorSubcoreMesh(core_axis_name="c", subcore_axis_name="s")

    @functools.partial(
        pl.kernel,
        mesh=mesh,
        compiler_params=pltpu.CompilerParams(needs_layout_passes=False),
        out_type=jax.ShapeDtypeStruct((2, 2, N_LOC, D_FEAT), jnp.float32),
        scratch_types=[
            pltpu.VMEM((CH_PT, CHUNK), jnp.int32),
            pltpu.VMEM((E_PT + CHUNK,), jnp.int32),
            pltpu.VMEM((CHUNK,), jnp.int32),
            pltpu.VMEM((CHUNK,), jnp.int32),
            pltpu.VMEM((CHUNK, D_FEAT), jnp.float32),
            pltpu.VMEM((N_LOC,), jnp.float32),
            pltpu.VMEM((ROWS_PT,), jnp.float32),
            pltpu.VMEM((ROWS_PT, D_FEAT), jnp.float32),
            pltpu.VMEM_SHARED((N_LOC, D_FEAT), jnp.float32),
            pltpu.VMEM_SHARED((16 * N_LOC,), jnp.float32),
            pltpu.SemaphoreType.DMA,
        ],
    )
    def k(x_h, edge_h, z_h, out_h,
          edge_v, pack_v, sloc_v, dloc_v, rows_v, hist_v, tbuf_v, sbuf_v,
          accum, stag, sem):
        c = lax.axis_index("c")
        s = lax.axis_index("s")
        zero16 = jnp.zeros((16,), jnp.float32)
        one16 = jnp.full((16,), 1.0, jnp.float32)

        pltpu.sync_copy(edge_h.at[s], edge_v)

        for p in range(2):
            base = (c * 2 + p) * QUARTER

            def zh(i, carry):
                hist_v[pl.ds(i * 16, 16)] = zero16
                return carry

            lax.fori_loop(0, N_LOC // 16, zh, 0)
            pltpu.sync_copy(z_h, accum.at[pl.ds(s * ROWS_PT, ROWS_PT)])
            plsc.subcore_barrier()

            def scan(i, off):
                for kk in range(CHUNK // 16):
                    sl = pl.ds(kk * 16, 16)
                    e16 = edge_v[i, sl]
                    d16 = lax.shift_right_logical(e16, SRC_BITS) - base
                    ok = (d16 >= 0) & (d16 < QUARTER)
                    dl = jnp.where(ok, d16, TRASH)
                    pk = (e16 & ((1 << SRC_BITS) - 1)) | (dl << SRC_BITS)
                    plsc.store_compressed(pack_v.at[pl.ds(off, 16)], pk,
                                          mask=ok)
                    plsc.addupdate_scatter(hist_v, [dl], one16, mask=ok)
                    cnt = plsc.all_reduce_population_count(ok)
                    off = off + cnt[0]
                return off

            off = lax.fori_loop(0, CH_PT, scan, jnp.int32(0))
            padv = jnp.full((16,), TRASH << SRC_BITS, jnp.int32)
            for kk in range(CHUNK // 16):
                pack_v[pl.ds(off + kk * 16, 16)] = padv
            nchunks = (off + CHUNK - 1) // CHUNK

            def body(i, carry):
                for kk in range(CHUNK // 16):
                    sl = pl.ds(kk * 16, 16)
                    pk = pack_v[pl.ds(i * CHUNK + kk * 16, 16)]
                    sloc_v[sl] = pk & ((1 << SRC_BITS) - 1)
                    dloc_v[sl] = lax.shift_right_logical(pk, SRC_BITS)
                pltpu.async_copy(x_h.at[sloc_v], rows_v, sem).wait()
                pltpu.sync_copy(rows_v, accum.at[dloc_v], add=True)
                return carry

            lax.fori_loop(0, nchunks, body, 0)

            pltpu.sync_copy(hist_v, stag.at[pl.ds(s * N_LOC, N_LOC)])
            plsc.subcore_barrier()

            def zt(i, carry):
                tbuf_v[pl.ds(i * 16, 16)] = zero16
                return carry

            lax.fori_loop(0, ROWS_PT // 16, zt, 0)
            for t in range(16):
                pltpu.sync_copy(
                    stag.at[pl.ds(t * N_LOC + s * ROWS_PT, ROWS_PT)],
                    hist_v.at[pl.ds(0, ROWS_PT)])

                def acc(i, carry):
                    tbuf_v[pl.ds(i * 16, 16)] = (tbuf_v[pl.ds(i * 16, 16)] +
                                                 hist_v[pl.ds(i * 16, 16)])
                    return carry

                lax.fori_loop(0, ROWS_PT // 16, acc, 0)

            def inv(i, carry):
                d = tbuf_v[pl.ds(i * 16, 16)]
                tbuf_v[pl.ds(i * 16, 16)] = 1.0 / jnp.maximum(d, 1.0)
                return carry

            lax.fori_loop(0, ROWS_PT // 16, inv, 0)

            pltpu.sync_copy(accum.at[pl.ds(s * ROWS_PT, ROWS_PT)], sbuf_v)

            def scale(rg, carry):
                iv16 = tbuf_v[pl.ds(rg * 16, 16)]
                for j in range(16):
                    r = rg * 16 + j
                    iv = iv16[j]
                    for kk in range(D_FEAT // 16):
                        sl = pl.ds(kk * 16, 16)
                        sbuf_v[r, sl] = sbuf_v[r, sl] * iv
                return carry

            lax.fori_loop(0, ROWS_PT // 16, scale, 0)
            pltpu.sync_copy(sbuf_v,
                            out_h.at[c, p, pl.ds(s * ROWS_PT, ROWS_PT)])
            plsc.subcore_barrier()

    return k(xfeat, edges3, zstripe)


def _encode_body(nm_ref, x_ref, w_ref, o_ref):
    wm = w_ref[...]
    w1 = wm[:, :D_FEAT]
    w2 = wm[:, D_FEAT:]
    h = lax.dot_general(x_ref[...], w1, (((1,), (1,)), ((), ())),
                        preferred_element_type=jnp.float32)
    h = h + lax.dot_general(nm_ref[...], w2, (((1,), (1,)), ((), ())),
                            preferred_element_type=jnp.float32)
    o_ref[...] = jnp.maximum(h, 0.0)


def _encode_tc(nm, x, W):
    br = 400
    grid = N_NODES // br
    return pl.pallas_call(
        _encode_body,
        grid=(grid,),
        in_specs=[
            pl.BlockSpec((br, D_FEAT), lambda i: (i, 0)),
            pl.BlockSpec((br, D_FEAT), lambda i: (i, 0)),
            pl.BlockSpec((D_FEAT, 2 * D_FEAT), lambda i: (0, 0)),
        ],
        out_specs=pl.BlockSpec((br, D_FEAT), lambda i: (i, 0)),
        out_shape=jax.ShapeDtypeStruct((N_NODES, D_FEAT), jnp.float32),
    )(nm, x, W)


def _gather_sc(ht, nuv3):
    mesh = plsc.VectorSubcoreMesh(core_axis_name="c", subcore_axis_name="s")
    rows_per_w = 2 * BATCH // 32

    @functools.partial(
        pl.kernel,
        mesh=mesh,
        out_type=jax.ShapeDtypeStruct((2 * BATCH, D_FEAT), jnp.float32),
        scratch_types=[
            pltpu.VMEM((rows_per_w // CHUNK, CHUNK), jnp.int32),
            pltpu.VMEM((rows_per_w, D_FEAT), jnp.float32),
            pltpu.SemaphoreType.DMA,
        ],
    )
    def k(ht_h, idx_h, out_h, idx_v, rows_v, sem):
        c = lax.axis_index("c")
        s = lax.axis_index("s")
        w = c * 16 + s
        pltpu.sync_copy(idx_h.at[w], idx_v)
        for j in range(rows_per_w // CHUNK):
            pltpu.async_copy(ht_h.at[idx_v.at[j]],
                             rows_v.at[pl.ds(j * CHUNK, CHUNK)], sem).wait()
        pltpu.sync_copy(rows_v, out_h.at[pl.ds(w * rows_per_w, rows_per_w)])

    return k(ht, nuv3)


def _cosine_body(u_ref, v_ref, o_ref):
    u = u_ref[...]
    v = v_ref[...]
    num = jnp.sum(u * v, axis=1, keepdims=True)
    uu = jnp.sum(u * u, axis=1, keepdims=True)
    vv = jnp.sum(v * v, axis=1, keepdims=True)
    o_ref[...] = num / jnp.maximum(jnp.sqrt(uu) * jnp.sqrt(vv), 1e-8)


def _cosine_tc(eu, ev):
    br = 512
    grid = BATCH // br
    return pl.pallas_call(
        _cosine_body,
        grid=(grid,),
        in_specs=[
            pl.BlockSpec((br, D_FEAT), lambda i: (i, 0)),
            pl.BlockSpec((br, D_FEAT), lambda i: (i, 0)),
        ],
        out_specs=pl.BlockSpec((br, 1), lambda i: (i, 0)),
        out_shape=jax.ShapeDtypeStruct((BATCH, 1), jnp.float32),
    )(eu, ev)


def kernel(x, edge_index, node_u, node_v, W):
    src = edge_index[0]
    dst = edge_index[1]
    pad = E_PAD - N_EDGES
    src_p = jnp.concatenate([src, jnp.zeros((pad,), jnp.int32)])
    dst_p = jnp.concatenate([dst, jnp.full((pad,), (1 << 17) - 1, jnp.int32)])
    edges3 = (src_p | (dst_p << SRC_BITS)).reshape(16, CH_PT, CHUNK)
    zstripe = jnp.zeros((ROWS_PT, D_FEAT), jnp.float32)

    nm_loc = _segment_mean_sc(x, edges3, zstripe)
    nm = jnp.concatenate([nm_loc[0, 0, :QUARTER], nm_loc[0, 1, :QUARTER],
                          nm_loc[1, 0, :QUARTER],
                          nm_loc[1, 1, :N_NODES - 3 * QUARTER]])
    assert 3 * QUARTER < N_NODES <= 4 * QUARTER
    ht = _encode_tc(nm, x, W)

    nuv3 = jnp.concatenate([node_u, node_v]).reshape(32, -1, CHUNK)
    euv = _gather_sc(ht, nuv3)
    cos2 = _cosine_tc(euv[:BATCH], euv[BATCH:])
    return cos2[:, 0]
